# baseline (speedup 1.0000x reference)
import jax
import jax.numpy as jnp
from jax.experimental import pallas as pl
from jax.experimental.pallas import tpu as pltpu

B, S, D = 8, 576, 256
Q, K = 4, 1024
N = B * S
TR = 512
NT = N // TR


def _rvq_body(x_ref, cb_ref, ct_ref, idx_ref, qout_ref, closs_ref):
    r = x_ref[...]
    qsum = jnp.zeros_like(r)
    iota = jax.lax.broadcasted_iota(jnp.int32, (TR, K), 1)
    closs_parts = []
    for q in range(Q):
        cb = cb_ref[q]
        ct = ct_ref[q]
        rnorm = jnp.sum(r * r, axis=1, keepdims=True)
        cnorm = jnp.sum(cb * cb, axis=1)
        dot = jax.lax.dot_general(r, ct, (((1,), (0,)), ((), ())),
                                  preferred_element_type=jnp.float32)
        d = rnorm - 2.0 * dot + cnorm[None, :]
        dmin = jnp.min(d, axis=1, keepdims=True)
        idx = jnp.min(jnp.where(d == dmin, iota, K), axis=1)
        idx_ref[q, :] = idx
        onehot = (iota == idx[:, None]).astype(jnp.float32)
        quant = jax.lax.dot_general(onehot, cb, (((1,), (0,)), ((), ())),
                                    preferred_element_type=jnp.float32,
                                    precision=jax.lax.Precision.HIGHEST)
        closs_parts.append(jnp.sum((quant - r) ** 2).reshape(1, 1))
        qsum = qsum + (r + (quant - r))
        r = r - quant
    qout_ref[...] = qsum
    closs_ref[...] = jnp.concatenate(closs_parts, axis=1)[None]


def kernel(x, codebooks):
    xr = x.reshape(N, D)
    ct = codebooks.transpose(0, 2, 1)
    idx_all, qout, closs = pl.pallas_call(
        _rvq_body,
        grid=(NT,),
        in_specs=[
            pl.BlockSpec((TR, D), lambda i: (i, 0)),
            pl.BlockSpec((Q, K, D), lambda i: (0, 0, 0)),
            pl.BlockSpec((Q, D, K), lambda i: (0, 0, 0)),
        ],
        out_specs=[
            pl.BlockSpec((Q, TR), lambda i: (0, i)),
            pl.BlockSpec((TR, D), lambda i: (i, 0)),
            pl.BlockSpec((1, 1, Q), lambda i: (i, 0, 0)),
        ],
        out_shape=[
            jax.ShapeDtypeStruct((Q, N), jnp.int32),
            jax.ShapeDtypeStruct((N, D), jnp.float32),
            jax.ShapeDtypeStruct((NT, 1, Q), jnp.float32),
        ],
    )(xr, codebooks, ct)
    all_indices = idx_all.T.reshape(B, S, Q)
    quantized_out = qout.reshape(B, S, D)
    commit_loss = closs.reshape(NT, Q).sum(axis=0) / (B * S * D)
    return (all_indices, quantized_out, commit_loss)


# speedup vs baseline: 1.4645x; 1.4645x over previous
import jax
import jax.numpy as jnp
from jax.experimental import pallas as pl

B, S, D = 8, 576, 256
Q, K = 4, 1024
N = B * S
TR = 512
NT = N // TR


def _rvq_body(x_ref, ct_ref, cn_ref, hi_ref, md_ref, lo_ref,
              idx_ref, qout_ref, closs_ref):
    r = x_ref[...]
    qsum = jnp.zeros_like(r)
    iota = jax.lax.broadcasted_iota(jnp.int32, (TR, K), 1)
    closs_parts = []
    for q in range(Q):
        ct = ct_ref[q]
        cnorm = cn_ref[q]
        rnorm = jnp.sum(r * r, axis=1, keepdims=True)
        dot = jax.lax.dot_general(r, ct, (((1,), (0,)), ((), ())),
                                  preferred_element_type=jnp.float32)
        d = rnorm - 2.0 * dot + cnorm
        dmin = jnp.min(d, axis=1, keepdims=True)
        idx = jnp.min(jnp.where(d == dmin, iota, K), axis=1)
        idx_ref[q, :] = idx
        onehot = (iota == idx[:, None]).astype(jnp.bfloat16)
        quant = jnp.float32(0.0)
        for part_ref in (hi_ref, md_ref, lo_ref):
            quant = quant + jax.lax.dot_general(
                onehot, part_ref[q], (((1,), (0,)), ((), ())),
                preferred_element_type=jnp.float32)
        closs_parts.append(jnp.sum((quant - r) ** 2).reshape(1, 1))
        qsum = qsum + (r + (quant - r))
        r = r - quant
    qout_ref[...] = qsum
    closs_ref[...] = jnp.concatenate(closs_parts, axis=1)[None]


def kernel(x, codebooks):
    xr = x.reshape(N, D)
    ct = codebooks.transpose(0, 2, 1)
    cn = jnp.sum(codebooks**2, axis=-1)[:, None, :]
    hi = codebooks.astype(jnp.bfloat16)
    r1 = codebooks - hi.astype(jnp.float32)
    md = r1.astype(jnp.bfloat16)
    lo = (r1 - md.astype(jnp.float32)).astype(jnp.bfloat16)
    idx_all, qout, closs = pl.pallas_call(
        _rvq_body,
        grid=(NT,),
        in_specs=[
            pl.BlockSpec((TR, D), lambda i: (i, 0)),
            pl.BlockSpec((Q, D, K), lambda i: (0, 0, 0)),
            pl.BlockSpec((Q, 1, K), lambda i: (0, 0, 0)),
            pl.BlockSpec((Q, K, D), lambda i: (0, 0, 0)),
            pl.BlockSpec((Q, K, D), lambda i: (0, 0, 0)),
            pl.BlockSpec((Q, K, D), lambda i: (0, 0, 0)),
        ],
        out_specs=[
            pl.BlockSpec((Q, TR), lambda i: (0, i)),
            pl.BlockSpec((TR, D), lambda i: (i, 0)),
            pl.BlockSpec((1, 1, Q), lambda i: (i, 0, 0)),
        ],
        out_shape=[
            jax.ShapeDtypeStruct((Q, N), jnp.int32),
            jax.ShapeDtypeStruct((N, D), jnp.float32),
            jax.ShapeDtypeStruct((NT, 1, Q), jnp.float32),
        ],
    )(xr, ct, cn, hi, md, lo)
    all_indices = idx_all.T.reshape(B, S, Q)
    quantized_out = qout.reshape(B, S, D)
    commit_loss = closs.reshape(NT, Q).sum(axis=0) / (B * S * D)
    return (all_indices, quantized_out, commit_loss)
